# Initial kernel scaffold; baseline (speedup 1.0000x reference)
import jax
import jax.numpy as jnp
from jax.experimental import pallas as pl


def kernel(inputs, initial_hidden_state, supports, Wg0, bg0, Wc0, bc0, Wg1, bg1, Wc1, bc1, Wp, bp):
    raise NotImplementedError("write your pallas kernel here")


# speedup vs baseline: 4.2226x; 4.2226x over previous
import jax
import jax.numpy as jnp
from jax.experimental import pallas as pl
from jax.experimental.pallas import tpu as pltpu

SEQ = 12
B = 32
N = 256
HID = 64
NM = 3
R = B * N

_F32 = jnp.float32
_BDIMS = (((2,), (1,)), ((0,), (0,)))


def _dcgru_step(s_ref, s2_ref, h0_ref,
                wg0_ref, bg0_ref, wc0x_ref, wc0h_ref, bc0_ref,
                wg1_ref, bg1_ref, wc1a_ref, wc1h_ref, bc1_ref,
                wp_ref, bp_ref,
                out_ref, g_s, q_s):
    t = pl.program_id(0)

    @pl.when(t == 0)
    def _init():
        g_s[:, 0:1] = jnp.zeros((R, 1), _F32)
        g_s[:, 1:65] = h0_ref[0]
        g_s[:, 65:] = jnp.zeros((R, 128 - 65), _F32)
        q_s[:, :HID] = jnp.zeros((R, HID), _F32)
        q_s[:, HID:] = h0_ref[1]

    S3 = jnp.broadcast_to(s_ref[...][None], (B, N, N))
    S23 = jnp.broadcast_to(s2_ref[...][None], (B, N, N))

    def cheb(x):
        f = x.shape[1]
        x3 = x.reshape(B, N, f)
        t1 = jax.lax.dot_general(S3, x3, _BDIMS, preferred_element_type=_F32)
        t2 = jax.lax.dot_general(S23, x3, _BDIMS, preferred_element_type=_F32)
        return t1.reshape(R, f), t2.reshape(R, f)

    def wsum(ts, w_ref):
        acc = jnp.dot(ts[0], w_ref[0], preferred_element_type=_F32)
        for m in (1, 2):
            acc = acc + jnp.dot(ts[m], w_ref[m], preferred_element_type=_F32)
        return acc

    def bmul(ts, w_ref):
        acc = ts[0] * w_ref[0]
        for m in (1, 2):
            acc = acc + ts[m] * w_ref[m]
        return acc

    g0 = g_s[:, :65]
    h0 = g0[:, 1:]
    g1, g2 = cheb(g0)
    gate = jax.nn.sigmoid(bg0_ref[...] + wsum((g0, g1, g2), wg0_ref))
    r = gate[:, :HID]
    u = gate[:, HID:]
    rh = r * h0
    rh1, rh2 = cheb(rh)
    tx = (g0[:, :1], g1[:, :1], g2[:, :1])
    c = jnp.tanh(bc0_ref[...] + bmul(tx, wc0x_ref)
                 + wsum((rh, rh1, rh2), wc0h_ref))
    h0n = u * h0 + (1.0 - u) * c
    g_s[:, 1:65] = h0n

    q_s[:, :HID] = h0n
    q0 = q_s[...]
    h1 = q0[:, HID:]
    q1, q2 = cheb(q0)
    gate1 = jax.nn.sigmoid(bg1_ref[...] + wsum((q0, q1, q2), wg1_ref))
    r1 = gate1[:, :HID]
    u1 = gate1[:, HID:]
    rr = r1 * h1
    rr1, rr2 = cheb(rr)
    ta = (q0[:, :HID], q1[:, :HID], q2[:, :HID])
    c1 = jnp.tanh(bc1_ref[...] + wsum(ta, wc1a_ref)
                  + wsum((rr, rr1, rr2), wc1h_ref))
    h1n = u1 * h1 + (1.0 - u1) * c1
    q_s[:, HID:] = h1n

    p = jnp.sum(h1n * wp_ref[...], axis=1, keepdims=True) + bp_ref[...]
    g_s[:, 0:1] = p
    for k in range(SEQ):
        @pl.when(t == k)
        def _store(k=k):
            out_ref[:, k:k + 1] = p


def kernel(inputs, initial_hidden_state, supports,
           Wg0, bg0, Wc0, bc0, Wg1, bg1, Wc1, bc1, Wp, bp):
    del inputs

    S = supports[0]
    S2 = 2.0 * (S @ S) - jnp.eye(N, dtype=S.dtype)

    h0 = initial_hidden_state.reshape(2, R, HID)

    wg0 = Wg0.reshape(1 + HID, NM, 2 * HID).transpose(1, 0, 2)
    wc0 = Wc0.reshape(1 + HID, NM, HID).transpose(1, 0, 2)
    wc0x = wc0[:, :1]
    wc0h = wc0[:, 1:]
    wg1 = Wg1.reshape(2 * HID, NM, 2 * HID).transpose(1, 0, 2)
    wc1 = Wc1.reshape(2 * HID, NM, HID).transpose(1, 0, 2)
    wc1a = wc1[:, :HID]
    wc1h = wc1[:, HID:]

    bg0_2 = bg0.reshape(1, 2 * HID)
    bc0_2 = bc0.reshape(1, HID)
    bg1_2 = bg1.reshape(1, 2 * HID)
    bc1_2 = bc1.reshape(1, HID)
    wp_row = Wp.reshape(1, HID)
    bp_2 = bp.reshape(1, 1)

    full = lambda shape: pl.BlockSpec(shape, lambda t: (0,) * len(shape))

    out = pl.pallas_call(
        _dcgru_step,
        grid=(SEQ,),
        in_specs=[
            full((N, N)), full((N, N)),
            full((2, R, HID)),
            full((NM, 1 + HID, 2 * HID)), full((1, 2 * HID)),
            full((NM, 1, HID)), full((NM, HID, HID)), full((1, HID)),
            full((NM, 2 * HID, 2 * HID)), full((1, 2 * HID)),
            full((NM, HID, HID)), full((NM, HID, HID)), full((1, HID)),
            full((1, HID)), full((1, 1)),
        ],
        out_specs=pl.BlockSpec((R, SEQ), lambda t: (0, 0)),
        out_shape=jax.ShapeDtypeStruct((R, SEQ), _F32),
        scratch_shapes=[
            pltpu.VMEM((R, 2 * HID), _F32),
            pltpu.VMEM((R, 2 * HID), _F32),
        ],
        compiler_params=pltpu.CompilerParams(
            dimension_semantics=("arbitrary",),
        ),
    )(S, S2, h0, wg0, bg0_2, wc0x, wc0h, bc0_2,
      wg1, bg1_2, wc1a, wc1h, bc1_2, wp_row, bp_2)

    return out.reshape(B, N, SEQ).transpose(2, 0, 1).reshape(SEQ, B, N)


# speedup vs baseline: 6.6741x; 1.5806x over previous
import jax
import jax.numpy as jnp
from jax.experimental import pallas as pl
from jax.experimental.pallas import tpu as pltpu

SEQ = 12
B = 32
N = 256
HID = 64
NM = 3
R = B * N

_F32 = jnp.float32
_BDIMS = (((2,), (1,)), ((0,), (0,)))


def _dcgru_step(ss_ref, h0_ref,
                wg0_ref, bg0_ref, wc0h_ref, bc0_ref,
                wg1_ref, bg1_ref, wc1h_ref, bc1_ref,
                wp_ref, bp_ref,
                out_ref, g_s, q_s):
    t = pl.program_id(0)

    @pl.when(t == 0)
    def _init():
        g_s[:, 0:1] = jnp.zeros((R, 1), _F32)
        g_s[:, 1:65] = h0_ref[0]
        g_s[:, 65:] = jnp.zeros((R, 128 - 65), _F32)
        q_s[:, :HID] = jnp.zeros((R, HID), _F32)
        q_s[:, HID:] = h0_ref[1]

    SS3 = jnp.broadcast_to(ss_ref[...][None], (B, 2 * N, N))

    def cheb(x):
        f = x.shape[1]
        x3 = x.reshape(B, N, f)
        y = jax.lax.dot_general(SS3, x3, _BDIMS, preferred_element_type=_F32)
        return y[:, :N, :].reshape(R, f), y[:, N:, :].reshape(R, f)

    def wsum(ts, w_ref):
        acc = jnp.dot(ts[0], w_ref[0], preferred_element_type=_F32)
        for m in (1, 2):
            acc = acc + jnp.dot(ts[m], w_ref[m], preferred_element_type=_F32)
        return acc

    g0 = g_s[:, :65]
    h0 = g0[:, 1:]
    g1, g2 = cheb(g0)
    big = wsum((g0, g1, g2), wg0_ref)
    gate = jax.nn.sigmoid(bg0_ref[...] + big[:, :2 * HID])
    r = gate[:, :HID]
    u = gate[:, HID:]
    rh = r * h0
    rh1, rh2 = cheb(rh)
    c = jnp.tanh(bc0_ref[...] + big[:, 2 * HID:]
                 + wsum((rh, rh1, rh2), wc0h_ref))
    h0n = u * h0 + (1.0 - u) * c
    g_s[:, 1:65] = h0n

    q_s[:, :HID] = h0n
    q0 = q_s[...]
    h1 = q0[:, HID:]
    q1, q2 = cheb(q0)
    big1 = wsum((q0, q1, q2), wg1_ref)
    gate1 = jax.nn.sigmoid(bg1_ref[...] + big1[:, :2 * HID])
    r1 = gate1[:, :HID]
    u1 = gate1[:, HID:]
    rr = r1 * h1
    rr1, rr2 = cheb(rr)
    c1 = jnp.tanh(bc1_ref[...] + big1[:, 2 * HID:]
                  + wsum((rr, rr1, rr2), wc1h_ref))
    h1n = u1 * h1 + (1.0 - u1) * c1
    q_s[:, HID:] = h1n

    p = jnp.dot(h1n, wp_ref[...], preferred_element_type=_F32) + bp_ref[...]
    g_s[:, 0:1] = p
    out_ref[0] = p


def kernel(inputs, initial_hidden_state, supports,
           Wg0, bg0, Wc0, bc0, Wg1, bg1, Wc1, bc1, Wp, bp):
    del inputs

    S = supports[0]
    S2 = 2.0 * (S @ S) - jnp.eye(N, dtype=S.dtype)
    SS = jnp.concatenate([S, S2], axis=0)

    h0 = initial_hidden_state.reshape(2, R, HID)

    wg0 = Wg0.reshape(1 + HID, NM, 2 * HID).transpose(1, 0, 2)
    wc0 = Wc0.reshape(1 + HID, NM, HID).transpose(1, 0, 2)
    wc0aug = jnp.concatenate(
        [wc0[:, :1], jnp.zeros((NM, HID, HID), _F32)], axis=1)
    wg0aug = jnp.concatenate([wg0, wc0aug], axis=2)
    wc0h = wc0[:, 1:]
    wg1 = Wg1.reshape(2 * HID, NM, 2 * HID).transpose(1, 0, 2)
    wc1 = Wc1.reshape(2 * HID, NM, HID).transpose(1, 0, 2)
    wc1aug = jnp.concatenate(
        [wc1[:, :HID], jnp.zeros((NM, HID, HID), _F32)], axis=1)
    wg1aug = jnp.concatenate([wg1, wc1aug], axis=2)
    wc1h = wc1[:, HID:]

    bg0_2 = bg0.reshape(1, 2 * HID)
    bc0_2 = bc0.reshape(1, HID)
    bg1_2 = bg1.reshape(1, 2 * HID)
    bc1_2 = bc1.reshape(1, HID)
    wp_col = Wp.reshape(HID, 1)
    bp_2 = bp.reshape(1, 1)

    full = lambda shape: pl.BlockSpec(shape, lambda t: (0,) * len(shape))

    out = pl.pallas_call(
        _dcgru_step,
        grid=(SEQ,),
        in_specs=[
            full((2 * N, N)),
            full((2, R, HID)),
            full((NM, 1 + HID, 3 * HID)), full((1, 2 * HID)),
            full((NM, HID, HID)), full((1, HID)),
            full((NM, 2 * HID, 3 * HID)), full((1, 2 * HID)),
            full((NM, HID, HID)), full((1, HID)),
            full((HID, 1)), full((1, 1)),
        ],
        out_specs=pl.BlockSpec((1, R, 1), lambda t: (t, 0, 0)),
        out_shape=jax.ShapeDtypeStruct((SEQ, R, 1), _F32),
        scratch_shapes=[
            pltpu.VMEM((R, 2 * HID), _F32),
            pltpu.VMEM((R, 2 * HID), _F32),
        ],
        compiler_params=pltpu.CompilerParams(
            dimension_semantics=("arbitrary",),
        ),
    )(SS, h0, wg0aug, bg0_2, wc0h, bc0_2,
      wg1aug, bg1_2, wc1h, bc1_2, wp_col, bp_2)

    return out.reshape(SEQ, B, N)


# speedup vs baseline: 8.7523x; 1.3114x over previous
import jax
import jax.numpy as jnp
from jax.experimental import pallas as pl
from jax.experimental.pallas import tpu as pltpu

SEQ = 12
B = 32
HB = 16
N = 256
HID = 64
NM = 3
R = B * N

_F32 = jnp.float32
_BF16 = jnp.bfloat16
_BDIMS = (((2,), (1,)), ((0,), (0,)))


def _dcgru_step(ss_ref, h0_ref,
                wg0_ref, bg0_ref, wc0h_ref, bc0_ref,
                wg1_ref, bg1_ref, wc1h_ref, bc1_ref,
                wp_ref, bp_ref,
                out_ref, g_s, q_s):
    t = pl.program_id(0)

    @pl.when(t == 0)
    def _init():
        g_s[:, :HID] = h0_ref[0]
        g_s[:, HID:] = jnp.zeros((R, HID), _F32)
        q_s[...] = h0_ref[1]

    SS3 = jnp.broadcast_to(ss_ref[...][None], (HB, 2 * N, N))

    def cheb(x):
        f = x.shape[1]
        x3 = x.reshape(HB, N, f)
        y = jax.lax.dot_general(SS3, x3.astype(_BF16), _BDIMS,
                                preferred_element_type=_F32)
        return y[:, :N, :].reshape(HB * N, f), y[:, N:, :].reshape(HB * N, f)

    def wsum(ts, w_ref):
        acc = jnp.dot(ts[0].astype(_BF16), w_ref[0], preferred_element_type=_F32)
        for m in (1, 2):
            acc = acc + jnp.dot(ts[m].astype(_BF16), w_ref[m],
                                preferred_element_type=_F32)
        return acc

    rows_a = slice(0, HB * N)
    rows_b = slice(HB * N, R)

    def l0_gate(g0):
        g1, g2 = cheb(g0)
        big = wsum((g0, g1, g2), wg0_ref)
        gate = jax.nn.sigmoid(bg0_ref[...] + big[:, :2 * HID])
        rh = gate[:, :HID] * g0[:, :HID]
        return big, gate[:, HID:], rh

    def l0_cand(big, u, rh, h0, rows):
        rh1, rh2 = cheb(rh)
        c = jnp.tanh(bc0_ref[...] + big[:, 2 * HID:]
                     + wsum((rh, rh1, rh2), wc0h_ref))
        h0n = u * h0 + (1.0 - u) * c
        g_s[rows, :HID] = h0n
        return jnp.concatenate([h0n, q_s[rows, :]], axis=1)

    def l1_gate(q0):
        q1, q2 = cheb(q0)
        big1 = wsum((q0, q1, q2), wg1_ref)
        gate1 = jax.nn.sigmoid(bg1_ref[...] + big1[:, :2 * HID])
        rr = gate1[:, :HID] * q0[:, HID:]
        return big1, gate1[:, HID:], rr

    def l1_cand(big1, u1, rr, h1, rows):
        rr1, rr2 = cheb(rr)
        c1 = jnp.tanh(bc1_ref[...] + big1[:, 2 * HID:]
                      + wsum((rr, rr1, rr2), wc1h_ref))
        h1n = u1 * h1 + (1.0 - u1) * c1
        q_s[rows, :] = h1n
        p = jnp.dot(h1n.astype(_BF16), wp_ref[...],
                    preferred_element_type=_F32) + bp_ref[...]
        g_s[rows, HID:HID + 1] = p
        out_ref[0, rows, :] = p

    g0a = g_s[rows_a, :HID + 1]
    g0b = g_s[rows_b, :HID + 1]
    biga, ua, rha = l0_gate(g0a)
    bigb, ub, rhb = l0_gate(g0b)
    q0a = l0_cand(biga, ua, rha, g0a[:, :HID], rows_a)
    q0b = l0_cand(bigb, ub, rhb, g0b[:, :HID], rows_b)
    big1a, u1a, rra = l1_gate(q0a)
    big1b, u1b, rrb = l1_gate(q0b)
    l1_cand(big1a, u1a, rra, q0a[:, HID:], rows_a)
    l1_cand(big1b, u1b, rrb, q0b[:, HID:], rows_b)


def kernel(inputs, initial_hidden_state, supports,
           Wg0, bg0, Wc0, bc0, Wg1, bg1, Wc1, bc1, Wp, bp):
    del inputs

    S = supports[0]
    S2 = 2.0 * (S @ S) - jnp.eye(N, dtype=S.dtype)
    SS = jnp.concatenate([S, S2], axis=0)

    h0 = initial_hidden_state.reshape(2, R, HID)

    wg0 = Wg0.reshape(1 + HID, NM, 2 * HID).transpose(1, 0, 2)
    wc0 = Wc0.reshape(1 + HID, NM, HID).transpose(1, 0, 2)
    wc0aug = jnp.concatenate(
        [wc0[:, :1], jnp.zeros((NM, HID, HID), _F32)], axis=1)
    wg0aug = jnp.concatenate([wg0, wc0aug], axis=2)
    wg0aug = jnp.concatenate([wg0aug[:, 1:], wg0aug[:, :1]], axis=1)
    wc0h = wc0[:, 1:]
    wg1 = Wg1.reshape(2 * HID, NM, 2 * HID).transpose(1, 0, 2)
    wc1 = Wc1.reshape(2 * HID, NM, HID).transpose(1, 0, 2)
    wc1aug = jnp.concatenate(
        [wc1[:, :HID], jnp.zeros((NM, HID, HID), _F32)], axis=1)
    wg1aug = jnp.concatenate([wg1, wc1aug], axis=2)
    wc1h = wc1[:, HID:]

    bg0_2 = bg0.reshape(1, 2 * HID)
    bc0_2 = bc0.reshape(1, HID)
    bg1_2 = bg1.reshape(1, 2 * HID)
    bc1_2 = bc1.reshape(1, HID)
    wp_col = Wp.reshape(HID, 1)
    bp_2 = bp.reshape(1, 1)

    full = lambda shape: pl.BlockSpec(shape, lambda t: (0,) * len(shape))

    out = pl.pallas_call(
        _dcgru_step,
        grid=(SEQ,),
        in_specs=[
            full((2 * N, N)),
            full((2, R, HID)),
            full((NM, 1 + HID, 3 * HID)), full((1, 2 * HID)),
            full((NM, HID, HID)), full((1, HID)),
            full((NM, 2 * HID, 3 * HID)), full((1, 2 * HID)),
            full((NM, HID, HID)), full((1, HID)),
            full((HID, 1)), full((1, 1)),
        ],
        out_specs=pl.BlockSpec((1, R, 1), lambda t: (t, 0, 0)),
        out_shape=jax.ShapeDtypeStruct((SEQ, R, 1), _F32),
        scratch_shapes=[
            pltpu.VMEM((R, 2 * HID), _F32),
            pltpu.VMEM((R, HID), _F32),
        ],
        compiler_params=pltpu.CompilerParams(
            dimension_semantics=("arbitrary",),
        ),
    )(SS.astype(_BF16), h0, wg0aug.astype(_BF16), bg0_2,
      wc0h.astype(_BF16), bc0_2, wg1aug.astype(_BF16), bg1_2,
      wc1h.astype(_BF16), bc1_2, wp_col.astype(_BF16), bp_2)

    return out.reshape(SEQ, B, N)


# speedup vs baseline: 9.2512x; 1.0570x over previous
import jax
import jax.numpy as jnp
from jax.experimental import pallas as pl
from jax.experimental.pallas import tpu as pltpu

SEQ = 12
B = 32
HB = 16
N = 256
HID = 64
NM = 3
R = B * N

_F32 = jnp.float32
_BF16 = jnp.bfloat16
_BDIMS = (((2,), (1,)), ((0,), (0,)))


def _dcgru_step(ss_ref, h0_ref,
                wg0_ref, bg0_ref, wc0h_ref, bc0_ref,
                wg1_ref, bg1_ref, wc1h_ref, bc1_ref,
                wp_ref, bp_ref,
                out_ref, g_s, q_s):
    t = pl.program_id(0)

    @pl.when(t == 0)
    def _init():
        g_s[:, :HID] = h0_ref[0]
        g_s[:, HID:] = jnp.zeros((R, HID), _F32)
        q_s[...] = h0_ref[1]

    SS3 = jnp.broadcast_to(ss_ref[...][None], (HB, 2 * N, N))

    def cheb(x):
        f = x.shape[1]
        x3 = x.reshape(HB, N, f)
        y = jax.lax.dot_general(SS3, x3.astype(_BF16), _BDIMS,
                                preferred_element_type=_F32)
        return y[:, :N, :].reshape(HB * N, f), y[:, N:, :].reshape(HB * N, f)

    def wsum(ts, w_ref):
        acc = jnp.dot(ts[0].astype(_BF16), w_ref[0], preferred_element_type=_F32)
        for m in (1, 2):
            acc = acc + jnp.dot(ts[m].astype(_BF16), w_ref[m],
                                preferred_element_type=_F32)
        return acc

    rows_a = slice(0, HB * N)
    rows_b = slice(HB * N, R)

    def l0_gate(g0):
        g1, g2 = cheb(g0)
        big = wsum((g0, g1, g2), wg0_ref)
        gate = jax.nn.sigmoid(bg0_ref[...] + big[:, :2 * HID])
        rh = gate[:, :HID] * g0[:, :HID]
        return big, gate[:, HID:], rh

    def l0_cand(big, u, rh, h0, rows):
        rh1, rh2 = cheb(rh)
        c = jnp.tanh(bc0_ref[...] + big[:, 2 * HID:]
                     + wsum((rh, rh1, rh2), wc0h_ref))
        h0n = c + u * (h0 - c)
        g_s[rows, :HID] = h0n
        return jnp.concatenate([h0n, q_s[rows, :]], axis=1)

    def l1_gate(q0):
        q1, q2 = cheb(q0)
        big1 = wsum((q0, q1, q2), wg1_ref)
        gate1 = jax.nn.sigmoid(bg1_ref[...] + big1[:, :2 * HID])
        rr = gate1[:, :HID] * q0[:, HID:]
        return big1, gate1[:, HID:], rr

    def l1_cand(big1, u1, rr, h1, rows, b0):
        rr1, rr2 = cheb(rr)
        c1 = jnp.tanh(bc1_ref[...] + big1[:, 2 * HID:]
                      + wsum((rr, rr1, rr2), wc1h_ref))
        h1n = c1 + u1 * (h1 - c1)
        q_s[rows, :] = h1n
        h1nb = h1n.astype(_BF16)
        p = jnp.dot(h1nb, wp_ref[...],
                    preferred_element_type=_F32) + bp_ref[...]
        g_s[rows, HID:HID + 1] = p
        p2 = jax.lax.dot_general(
            jnp.broadcast_to(wp_ref[...][None], (HB, HID, 1)),
            h1nb.reshape(HB, N, HID), (((1,), (2,)), ((0,), (0,))),
            preferred_element_type=_F32)
        out_ref[0, b0:b0 + HB, :] = p2.reshape(HB, N) + bp_ref[...]

    g0a = g_s[rows_a, :HID + 1]
    g0b = g_s[rows_b, :HID + 1]
    biga, ua, rha = l0_gate(g0a)
    bigb, ub, rhb = l0_gate(g0b)
    q0a = l0_cand(biga, ua, rha, g0a[:, :HID], rows_a)
    q0b = l0_cand(bigb, ub, rhb, g0b[:, :HID], rows_b)
    big1a, u1a, rra = l1_gate(q0a)
    big1b, u1b, rrb = l1_gate(q0b)
    l1_cand(big1a, u1a, rra, q0a[:, HID:], rows_a, 0)
    l1_cand(big1b, u1b, rrb, q0b[:, HID:], rows_b, HB)


def kernel(inputs, initial_hidden_state, supports,
           Wg0, bg0, Wc0, bc0, Wg1, bg1, Wc1, bc1, Wp, bp):
    del inputs

    S = supports[0]
    S2 = 2.0 * (S @ S) - jnp.eye(N, dtype=S.dtype)
    SS = jnp.concatenate([S, S2], axis=0)

    h0 = initial_hidden_state.reshape(2, R, HID)

    wg0 = Wg0.reshape(1 + HID, NM, 2 * HID).transpose(1, 0, 2)
    wc0 = Wc0.reshape(1 + HID, NM, HID).transpose(1, 0, 2)
    wc0aug = jnp.concatenate(
        [wc0[:, :1], jnp.zeros((NM, HID, HID), _F32)], axis=1)
    wg0aug = jnp.concatenate([wg0, wc0aug], axis=2)
    wg0aug = jnp.concatenate([wg0aug[:, 1:], wg0aug[:, :1]], axis=1)
    wc0h = wc0[:, 1:]
    wg1 = Wg1.reshape(2 * HID, NM, 2 * HID).transpose(1, 0, 2)
    wc1 = Wc1.reshape(2 * HID, NM, HID).transpose(1, 0, 2)
    wc1aug = jnp.concatenate(
        [wc1[:, :HID], jnp.zeros((NM, HID, HID), _F32)], axis=1)
    wg1aug = jnp.concatenate([wg1, wc1aug], axis=2)
    wc1h = wc1[:, HID:]

    bg0_2 = bg0.reshape(1, 2 * HID)
    bc0_2 = bc0.reshape(1, HID)
    bg1_2 = bg1.reshape(1, 2 * HID)
    bc1_2 = bc1.reshape(1, HID)
    wp_col = Wp.reshape(HID, 1)
    bp_2 = bp.reshape(1, 1)

    full = lambda shape: pl.BlockSpec(shape, lambda t: (0,) * len(shape))

    out = pl.pallas_call(
        _dcgru_step,
        grid=(SEQ,),
        in_specs=[
            full((2 * N, N)),
            full((2, R, HID)),
            full((NM, 1 + HID, 3 * HID)), full((1, 2 * HID)),
            full((NM, HID, HID)), full((1, HID)),
            full((NM, 2 * HID, 3 * HID)), full((1, 2 * HID)),
            full((NM, HID, HID)), full((1, HID)),
            full((HID, 1)), full((1, 1)),
        ],
        out_specs=pl.BlockSpec((1, B, N), lambda t: (t, 0, 0)),
        out_shape=jax.ShapeDtypeStruct((SEQ, B, N), _F32),
        scratch_shapes=[
            pltpu.VMEM((R, 2 * HID), _F32),
            pltpu.VMEM((R, HID), _F32),
        ],
        compiler_params=pltpu.CompilerParams(
            dimension_semantics=("arbitrary",),
        ),
    )(SS.astype(_BF16), h0, wg0aug.astype(_BF16), bg0_2,
      wc0h.astype(_BF16), bc0_2, wg1aug.astype(_BF16), bg1_2,
      wc1h.astype(_BF16), bc1_2, wp_col.astype(_BF16), bp_2)

    return out
